# baseline (speedup 1.0000x reference)
import functools
import math

import jax
import jax.numpy as jnp
from jax import lax
from jax.experimental import pallas as pl
from jax.experimental.pallas import tpu as pltpu
from jax.experimental.pallas import tpu_sc as plsc

S = 2048
E = 768
V = 8192
H = 12
HS = 64
EX = 8
FF = 3072
LAYERS = 2

BT = 256
NBLK = S // BT
CAP = S * 2 + EX * BT
GBLK = CAP // BT
NEG = -3.0e38


def _ln_rows(x, g, b):
    m = jnp.mean(x, axis=1, keepdims=True)
    xc = x - m
    var = jnp.mean(xc * xc, axis=1, keepdims=True)
    return xc * lax.rsqrt(var + 1e-5) * g + b


def _add2_body(a_ref, b_ref, o_ref):
    o_ref[...] = a_ref[...] + b_ref[...]


def _add2(a, b):
    return pl.pallas_call(
        _add2_body,
        grid=(NBLK,),
        in_specs=[pl.BlockSpec((BT, E), lambda i: (i, 0)),
                  pl.BlockSpec((BT, E), lambda i: (i, 0))],
        out_specs=pl.BlockSpec((BT, E), lambda i: (i, 0)),
        out_shape=jax.ShapeDtypeStruct((S, E), jnp.float32),
    )(a, b)


def _qkv_body(x_ref, g_ref, b_ref, wq_ref, wk_ref, wv_ref, bq_ref, bk_ref,
              bv_ref, q_ref, k_ref, v_ref):
    h = _ln_rows(x_ref[...], g_ref[...], b_ref[...])
    q_ref[...] = jnp.dot(h, wq_ref[...], preferred_element_type=jnp.float32) + bq_ref[...]
    k_ref[...] = jnp.dot(h, wk_ref[...], preferred_element_type=jnp.float32) + bk_ref[...]
    v_ref[...] = jnp.dot(h, wv_ref[...], preferred_element_type=jnp.float32) + bv_ref[...]


def _qkv(x, g, b, wq, wk, wv, bq, bk, bv):
    row = pl.BlockSpec((BT, E), lambda i: (i, 0))
    full = pl.BlockSpec((E, E), lambda i: (0, 0))
    vec = pl.BlockSpec((1, E), lambda i: (0, 0))
    return pl.pallas_call(
        _qkv_body,
        grid=(NBLK,),
        in_specs=[row, vec, vec, full, full, full, vec, vec, vec],
        out_specs=[row, row, row],
        out_shape=[jax.ShapeDtypeStruct((S, E), jnp.float32)] * 3,
    )(x, g, b, wq, wk, wv, bq, bk, bv)


def _attn_body(q_ref, k_ref, v_ref, o_ref):
    i = pl.program_id(1)
    qb = q_ref[0].astype(jnp.bfloat16)
    rows = i * BT + lax.broadcasted_iota(jnp.int32, (BT, BT), 0)
    cols = lax.broadcasted_iota(jnp.int32, (BT, BT), 1)

    def body(j, carry):
        acc, m_run, l_run = carry
        kb = k_ref[0, pl.ds(j * BT, BT), :].astype(jnp.bfloat16)
        vb = v_ref[0, pl.ds(j * BT, BT), :].astype(jnp.bfloat16)
        s = lax.dot_general(qb, kb, (((1,), (1,)), ((), ())),
                            preferred_element_type=jnp.float32)
        s = s * (1.0 / math.sqrt(E))
        s = jnp.where(j * BT + cols <= rows, s, NEG)
        m_new = jnp.maximum(m_run, jnp.max(s, axis=1, keepdims=True))
        alpha = jnp.exp(m_run - m_new)
        p = jnp.exp(s - m_new)
        l_new = l_run * alpha + jnp.sum(p, axis=1, keepdims=True)
        pv = jnp.dot(p.astype(jnp.bfloat16), vb,
                     preferred_element_type=jnp.float32)
        return acc * alpha + pv, m_new, l_new

    acc0 = (jnp.zeros((BT, HS), jnp.float32),
            jnp.full((BT, 1), NEG, jnp.float32),
            jnp.zeros((BT, 1), jnp.float32))
    acc, _, l_run = lax.fori_loop(0, i + 1, body, acc0)
    o_ref[0] = acc / l_run


def _attn(q, k, v):
    qspec = pl.BlockSpec((1, BT, HS), lambda h, i: (h, i, 0))
    kspec = pl.BlockSpec((1, S, HS), lambda h, i: (h, 0, 0))
    return pl.pallas_call(
        _attn_body,
        grid=(H, NBLK),
        in_specs=[qspec, kspec, kspec],
        out_specs=qspec,
        out_shape=jax.ShapeDtypeStruct((H, S, HS), jnp.float32),
    )(q, k, v)


def _outproj_body(o_ref, wo_ref, bo_ref, xr_ref, g2_ref, b2_ref, rw_ref,
                  rb_ref, x1_ref, y_ref, rs_ref):
    x1 = (jnp.dot(o_ref[...], wo_ref[...], preferred_element_type=jnp.float32)
          + bo_ref[...] + xr_ref[...])
    x1_ref[...] = x1
    y = _ln_rows(x1, g2_ref[...], b2_ref[...])
    y_ref[...] = y
    rs_ref[...] = jnp.dot(y, rw_ref[...], preferred_element_type=jnp.float32) + rb_ref[...]


def _outproj(o, wo, bo, xr, g2, b2, rw, rb):
    row = pl.BlockSpec((BT, E), lambda i: (i, 0))
    full = pl.BlockSpec((E, E), lambda i: (0, 0))
    vec = pl.BlockSpec((1, E), lambda i: (0, 0))
    return pl.pallas_call(
        _outproj_body,
        grid=(NBLK,),
        in_specs=[row, full, vec, row, vec, vec,
                  pl.BlockSpec((E, EX), lambda i: (0, 0)),
                  pl.BlockSpec((1, EX), lambda i: (0, 0))],
        out_specs=[row, row, pl.BlockSpec((BT, EX), lambda i: (i, 0))],
        out_shape=[jax.ShapeDtypeStruct((S, E), jnp.float32),
                   jax.ShapeDtypeStruct((S, E), jnp.float32),
                   jax.ShapeDtypeStruct((S, EX), jnp.float32)],
    )(o, wo, bo, xr, g2, b2, rw, rb)


def _ffn_body(be_ref, ba_ref, y_ref, w1_ref, b1_ref, w2_ref, b2_ref, out_ref):
    b = pl.program_id(0)

    @pl.when(ba_ref[b] == 1)
    def _():
        y = y_ref[...].astype(jnp.bfloat16)
        h = jnp.dot(y, w1_ref[0], preferred_element_type=jnp.float32) + b1_ref[0]
        h = jnp.maximum(h, 0.0).astype(jnp.bfloat16)
        out_ref[...] = jnp.dot(h, w2_ref[0], preferred_element_type=jnp.float32) + b2_ref[0]


def _ffn(be, ba, y_rows, ew1, eb1, ew2, eb2):
    grid_spec = pltpu.PrefetchScalarGridSpec(
        num_scalar_prefetch=2,
        grid=(GBLK,),
        in_specs=[
            pl.BlockSpec((BT, E), lambda b, be, ba: (jnp.where(ba[b] == 1, b, 0), 0)),
            pl.BlockSpec((1, E, FF), lambda b, be, ba: (be[b], 0, 0)),
            pl.BlockSpec((1, 1, FF), lambda b, be, ba: (be[b], 0, 0)),
            pl.BlockSpec((1, FF, E), lambda b, be, ba: (be[b], 0, 0)),
            pl.BlockSpec((1, 1, E), lambda b, be, ba: (be[b], 0, 0)),
        ],
        out_specs=pl.BlockSpec((BT, E), lambda b, be, ba: (b, 0)),
    )
    return pl.pallas_call(
        _ffn_body,
        grid_spec=grid_spec,
        out_shape=jax.ShapeDtypeStruct((CAP, E), jnp.float32),
    )(be, ba, y_rows, ew1, eb1, ew2, eb2)


def _combine_body(x1_ref, g1_ref, g2_ref, w1_ref, w2_ref, o_ref):
    w1 = w1_ref[:, 0:1]
    w2 = w2_ref[:, 0:1]
    o_ref[...] = x1_ref[...] + g1_ref[...] * w1 + g2_ref[...] * w2


def _combine(x1, g1, g2, w1b, w2b):
    row = pl.BlockSpec((BT, E), lambda i: (i, 0))
    wspec = pl.BlockSpec((BT, 128), lambda i: (i, 0))
    return pl.pallas_call(
        _combine_body,
        grid=(NBLK,),
        in_specs=[row, row, row, wspec, wspec],
        out_specs=row,
        out_shape=jax.ShapeDtypeStruct((S, E), jnp.float32),
    )(x1, g1, g2, w1b, w2b)


def _lnf_body(x_ref, g_ref, b_ref, o_ref):
    o_ref[...] = _ln_rows(x_ref[...], g_ref[...], b_ref[...])


def _lnf(x, g, b):
    row = pl.BlockSpec((BT, E), lambda i: (i, 0))
    vec = pl.BlockSpec((1, E), lambda i: (0, 0))
    return pl.pallas_call(
        _lnf_body,
        grid=(NBLK,),
        in_specs=[row, vec, vec],
        out_specs=row,
        out_shape=jax.ShapeDtypeStruct((S, E), jnp.float32),
    )(x, g, b)


_BR = 512
_BV = 512


def _lmhead_body(x_ref, w_ref, b_ref, o_ref):
    xb = x_ref[...].astype(jnp.bfloat16)
    o_ref[...] = (jnp.dot(xb, w_ref[...], preferred_element_type=jnp.float32)
                  + b_ref[...])


def _lmhead(x, w, b):
    return pl.pallas_call(
        _lmhead_body,
        grid=(S // _BR, V // _BV),
        in_specs=[pl.BlockSpec((_BR, E), lambda i, j: (i, 0)),
                  pl.BlockSpec((E, _BV), lambda i, j: (0, j)),
                  pl.BlockSpec((1, _BV), lambda i, j: (0, j))],
        out_specs=pl.BlockSpec((_BR, _BV), lambda i, j: (i, j)),
        out_shape=jax.ShapeDtypeStruct((S, V), jnp.float32),
    )(x, w, b)


@functools.lru_cache(maxsize=None)
def _make_sc_gather(nrows_table, ncols, nrows_out):
    info = plsc.get_sparse_core_info()
    nw = info.num_cores * info.num_subcores
    b_per_w = nrows_out // nw
    ch = min(b_per_w, 64)
    n_chunks = b_per_w // ch
    mesh = plsc.VectorSubcoreMesh(core_axis_name="c", subcore_axis_name="s")

    @functools.partial(
        pl.kernel, mesh=mesh,
        out_type=jax.ShapeDtypeStruct((nrows_out, ncols), jnp.float32),
        compiler_params=pltpu.CompilerParams(needs_layout_passes=False),
        scratch_types=[pltpu.VMEM((ch,), jnp.int32),
                       pltpu.VMEM((ch, ncols), jnp.float32),
                       pltpu.SemaphoreType.DMA],
    )
    def k(table_hbm, idx_hbm, out_hbm, idx_v, rows_v, sem):
        wid = lax.axis_index("s") * info.num_cores + lax.axis_index("c")
        base = wid * b_per_w
        for c in range(n_chunks):
            off = base + c * ch
            pltpu.sync_copy(idx_hbm.at[pl.ds(off, ch)], idx_v)
            pltpu.async_copy(table_hbm.at[idx_v], rows_v, sem).wait()
            pltpu.sync_copy(rows_v, out_hbm.at[pl.ds(off, ch)])

    return k


def _sc_gather(table, idx):
    k = _make_sc_gather(table.shape[0], table.shape[1], idx.shape[0])
    return k(table, idx)


def _sc_meta_body(rs_hbm, src_hbm, pp1_hbm, pp2_hbm, w1_hbm, w2_hbm, be_hbm,
                  ba_hbm, rsl, e1l, e2l, w1l, w2l, ecl, seg, crow, cuml, pp1l,
                  pp2l, z256, bel, bal, ec_s, cum_s, sem):
    core = lax.axis_index("c")
    s = lax.axis_index("s")

    @pl.when(core == 0)
    def _():
        iota = lax.broadcasted_iota(jnp.int32, (16,), 0)

        pltpu.sync_copy(rs_hbm, rsl)

        def pa(c, carry):
            gsl = pl.ds(s * 128 + c * 16, 16)
            r = [rsl[e, gsl] for e in range(EX)]
            m1 = r[0]
            for e in range(1, EX):
                m1 = jnp.maximum(m1, r[e])
            i1 = jnp.zeros((16,), jnp.int32)
            for e in range(EX - 1, -1, -1):
                i1 = jnp.where(r[e] == m1, e, i1)
            r2 = [jnp.where(i1 == e, NEG, r[e]) for e in range(EX)]
            m2 = r2[0]
            for e in range(1, EX):
                m2 = jnp.maximum(m2, r2[e])
            i2 = jnp.zeros((16,), jnp.int32)
            for e in range(EX - 1, -1, -1):
                i2 = jnp.where(r2[e] == m2, e, i2)
            ew = jnp.exp(m2 - m1)
            den = 1.0 + ew
            lsl = pl.ds(c * 16, 16)
            e1l[lsl] = i1
            e2l[lsl] = i2
            w1l[lsl] = 1.0 / den
            w2l[lsl] = ew / den
            return carry

        lax.fori_loop(0, 8, pa, jnp.int32(0))
        pltpu.sync_copy(w1l, w1_hbm.at[pl.ds(s * 128, 128)])
        pltpu.sync_copy(w2l, w2_hbm.at[pl.ds(s * 128, 128)])
        pltpu.sync_copy(e1l, ec_s.at[0, pl.ds(s * 128, 128)])
        pltpu.sync_copy(e2l, ec_s.at[1, pl.ds(s * 128, 128)])
        plsc.subcore_barrier()

        @pl.when(s < EX)
        def _b1():
            pltpu.sync_copy(ec_s, ecl)

            def cb(g, acc):
                rowvec = jnp.zeros((16,), jnp.int32)
                for cc in range(16):
                    jsl = pl.ds(g * 256 + cc * 16, 16)
                    rowvec = jnp.where(iota == cc, acc, rowvec)
                    m = (jnp.where(ecl[0, jsl] == s, 1, 0)
                         + jnp.where(ecl[1, jsl] == s, 1, 0))
                    acc = acc + jnp.sum(m)
                crow[pl.ds(g * 16, 16)] = rowvec
                return acc

            tot = lax.fori_loop(0, 8, cb, jnp.int32(0))
            crow[pl.ds(128, 16)] = jnp.zeros((16,), jnp.int32) + tot
            def zpad(i, c):
                crow[pl.ds(144 + i * 16, 16)] = jnp.zeros((16,), jnp.int32)
                return c

            lax.fori_loop(0, 7, zpad, jnp.int32(0))
            pltpu.sync_copy(crow, cum_s.at[0, pl.ds(s * 256, 256)])

        plsc.subcore_barrier()

        pltpu.sync_copy(cum_s, cuml)
        bases = []
        tots = []
        acc = jnp.int32(0)
        for e in range(EX):
            bases.append(acc)
            tote = cuml[0, pl.ds(e * 256 + 128, 16)][0]
            tots.append(tote)
            acc = acc + ((tote + (BT - 1)) // BT) * BT
        total = acc

        @pl.when(s < EX)
        def _b2():
            def zb(i, c):
                seg[pl.ds(i * 16, 16)] = jnp.zeros((16,), jnp.int32)
                return c

            lax.fori_loop(0, (S + 16) // 16, zb, jnp.int32(0))

            def sb(j, cur):
                tv = iota + j * 16
                jsl = pl.ds(j * 16, 16)
                m1 = ecl[0, jsl] == s
                plsc.store_compressed(seg.at[pl.ds(cur, 16)], tv, mask=m1)
                cur = cur + jnp.sum(jnp.where(m1, 1, 0))
                m2 = ecl[1, jsl] == s
                plsc.store_compressed(seg.at[pl.ds(cur, 16)], tv, mask=m2)
                cur = cur + jnp.sum(jnp.where(m2, 1, 0))
                return cur

            tot_self = lax.fori_loop(0, 128, sb, jnp.int32(0))
            mybase = jnp.int32(0)
            for e in range(EX):
                mybase = jnp.where(s == e, bases[e], mybase)
            for i in range(EX):
                @pl.when(i * BT < tot_self)
                def _cp(i=i):
                    off = pl.multiple_of(mybase + i * BT, BT)
                    pltpu.sync_copy(seg.at[pl.ds(i * BT, BT)],
                                    src_hbm.at[pl.ds(off, BT)])

        @pl.when(s == EX)
        def _tz():
            for i in range(16):
                z256[pl.ds(i * 16, 16)] = jnp.zeros((16,), jnp.int32)
            for i in range(GBLK):
                @pl.when(i * BT >= total)
                def _z(i=i):
                    pltpu.sync_copy(z256, src_hbm.at[pl.ds(i * BT, BT)])

        @pl.when(s == EX + 1)
        def _tb():
            for jb in range(2):
                bv = iota + jb * 16
                rowstart = bv * BT
                bex = jnp.zeros((16,), jnp.int32)
                for e in range(1, EX):
                    bex = jnp.where(rowstart >= bases[e], e, bex)
                bact = jnp.where(rowstart < total, 1, 0)
                bel[pl.ds(jb * 16, 16)] = bex
                bal[pl.ds(jb * 16, 16)] = bact
            pltpu.sync_copy(bel, be_hbm)
            pltpu.sync_copy(bal, ba_hbm)

        def pc(c, carry):
            lane = (s % 2) * 8 + c
            blk = (s // 2) * 16
            lsl = pl.ds(c * 16, 16)
            e1 = e1l[lsl]
            e2 = e2l[lsl]
            pos1 = jnp.zeros((16,), jnp.int32)
            pos2 = jnp.zeros((16,), jnp.int32)
            for e in range(EX):
                m1 = e1 == e
                mi1 = jnp.where(m1, 1, 0)
                cs1 = plsc.cumsum(mi1)
                n1 = jnp.sum(mi1)
                m2 = e2 == e
                mi2 = jnp.where(m2, 1, 0)
                cs2 = plsc.cumsum(mi2)
                cumv = cuml[0, pl.ds(e * 256 + blk, 16)]
                cumej = jnp.sum(jnp.where(iota == lane, cumv, 0))
                start = bases[e] + cumej
                pos1 = jnp.where(m1, start + cs1 - 1, pos1)
                pos2 = jnp.where(m2, start + n1 + cs2 - 1, pos2)
            pp1l[lsl] = pos1
            pp2l[lsl] = pos2
            return carry

        lax.fori_loop(0, 8, pc, jnp.int32(0))
        pltpu.sync_copy(pp1l, pp1_hbm.at[pl.ds(s * 128, 128)])
        pltpu.sync_copy(pp2l, pp2_hbm.at[pl.ds(s * 128, 128)])


@functools.lru_cache(maxsize=None)
def _make_sc_meta():
    mesh = plsc.VectorSubcoreMesh(core_axis_name="c", subcore_axis_name="s")
    return functools.partial(
        pl.kernel, mesh=mesh,
        compiler_params=pltpu.CompilerParams(needs_layout_passes=False),
        out_type=[jax.ShapeDtypeStruct((CAP,), jnp.int32),
                  jax.ShapeDtypeStruct((S,), jnp.int32),
                  jax.ShapeDtypeStruct((S,), jnp.int32),
                  jax.ShapeDtypeStruct((S,), jnp.float32),
                  jax.ShapeDtypeStruct((S,), jnp.float32),
                  jax.ShapeDtypeStruct((32,), jnp.int32),
                  jax.ShapeDtypeStruct((32,), jnp.int32)],
        scratch_types=[pltpu.VMEM((EX, S), jnp.float32),
                       pltpu.VMEM((128,), jnp.int32),
                       pltpu.VMEM((128,), jnp.int32),
                       pltpu.VMEM((128,), jnp.float32),
                       pltpu.VMEM((128,), jnp.float32),
                       pltpu.VMEM((2, S), jnp.int32),
                       pltpu.VMEM((S + 16,), jnp.int32),
                       pltpu.VMEM((256,), jnp.int32),
                       pltpu.VMEM((1, EX * 256), jnp.int32),
                       pltpu.VMEM((128,), jnp.int32),
                       pltpu.VMEM((128,), jnp.int32),
                       pltpu.VMEM((BT,), jnp.int32),
                       pltpu.VMEM((32,), jnp.int32),
                       pltpu.VMEM((32,), jnp.int32),
                       pltpu.VMEM_SHARED((2, S), jnp.int32),
                       pltpu.VMEM_SHARED((1, EX * 256), jnp.int32),
                       pltpu.SemaphoreType.DMA],
    )(_sc_meta_body)


def _sc_meta(rs_t):
    return _make_sc_meta()(rs_t)


def kernel(inputs, tok_emb, pos_emb, ln1_g, ln1_b, wq, bq, wk, bk, wv, bv, wo,
           bo, ln2_g, ln2_b, router_w, router_b, e_w1, e_b1, e_w2, e_b2,
           lnf_g, lnf_b, out_w, out_b):
    ids = inputs.reshape(S).astype(jnp.int32)
    emb = _sc_gather(tok_emb, ids)
    x = _add2(emb, pos_emb)
    for l in range(LAYERS):
        wq_l = wq[l].transpose(1, 0, 2).reshape(E, H * HS)
        wk_l = wk[l].transpose(1, 0, 2).reshape(E, H * HS)
        wv_l = wv[l].transpose(1, 0, 2).reshape(E, H * HS)
        q, k, v = _qkv(x, ln1_g[l].reshape(1, E), ln1_b[l].reshape(1, E),
                       wq_l, wk_l, wv_l, bq[l].reshape(1, H * HS),
                       bk[l].reshape(1, H * HS), bv[l].reshape(1, H * HS))
        qh = q.reshape(S, H, HS).transpose(1, 0, 2)
        kh = k.reshape(S, H, HS).transpose(1, 0, 2)
        vh = v.reshape(S, H, HS).transpose(1, 0, 2)
        o = _attn(qh, kh, vh).transpose(1, 0, 2).reshape(S, H * HS)
        x1, y, rs = _outproj(o, wo[l], bo[l].reshape(1, E), x,
                             ln2_g[l].reshape(1, E), ln2_b[l].reshape(1, E),
                             router_w[l], router_b[l].reshape(1, EX))
        src_tok, pp1, pp2, w1, w2, be, ba = _sc_meta(rs.T)
        y_rows = _sc_gather(y, src_tok)
        out_g = _ffn(be, ba, y_rows, e_w1[l].astype(jnp.bfloat16),
                     e_b1[l].reshape(EX, 1, FF),
                     e_w2[l].astype(jnp.bfloat16),
                     e_b2[l].reshape(EX, 1, E))
        g1 = _sc_gather(out_g, pp1)
        g2 = _sc_gather(out_g, pp2)
        w1b = jnp.broadcast_to(w1[:, None], (S, 128))
        w2b = jnp.broadcast_to(w2[:, None], (S, 128))
        x = _combine(x1, g1, g2, w1b, w2b)
    xf = _lnf(x, lnf_g.reshape(1, E), lnf_b.reshape(1, E))
    logits = _lmhead(xf, out_w.astype(jnp.bfloat16), out_b.reshape(1, V))
    return logits.reshape(1, S, V)


# speedup vs baseline: 1.1488x; 1.1488x over previous
import functools
import math

import jax
import jax.numpy as jnp
from jax import lax
from jax.experimental import pallas as pl
from jax.experimental.pallas import tpu as pltpu
from jax.experimental.pallas import tpu_sc as plsc

S = 2048
E = 768
V = 8192
H = 12
HS = 64
EX = 8
FF = 3072
LAYERS = 2

BT = 256
NBLK = S // BT
CAP = S * 2 + EX * BT
GBLK = CAP // BT
NEG = -3.0e38


def _ln_rows(x, g, b):
    m = jnp.mean(x, axis=1, keepdims=True)
    xc = x - m
    var = jnp.mean(xc * xc, axis=1, keepdims=True)
    return xc * lax.rsqrt(var + 1e-5) * g + b


def _add2_body(a_ref, b_ref, o_ref):
    o_ref[...] = a_ref[...] + b_ref[...]


def _add2(a, b):
    return pl.pallas_call(
        _add2_body,
        grid=(NBLK,),
        in_specs=[pl.BlockSpec((BT, E), lambda i: (i, 0)),
                  pl.BlockSpec((BT, E), lambda i: (i, 0))],
        out_specs=pl.BlockSpec((BT, E), lambda i: (i, 0)),
        out_shape=jax.ShapeDtypeStruct((S, E), jnp.float32),
    )(a, b)


def _qkv_body(x_ref, g_ref, b_ref, wq_ref, wk_ref, wv_ref, bq_ref, bk_ref,
              bv_ref, q_ref, k_ref, v_ref):
    h = _ln_rows(x_ref[...], g_ref[...], b_ref[...])
    q_ref[...] = jnp.dot(h, wq_ref[...], preferred_element_type=jnp.float32) + bq_ref[...]
    k_ref[...] = jnp.dot(h, wk_ref[...], preferred_element_type=jnp.float32) + bk_ref[...]
    v_ref[...] = jnp.dot(h, wv_ref[...], preferred_element_type=jnp.float32) + bv_ref[...]


def _qkv(x, g, b, wq, wk, wv, bq, bk, bv):
    row = pl.BlockSpec((BT, E), lambda i: (i, 0))
    full = pl.BlockSpec((E, E), lambda i: (0, 0))
    vec = pl.BlockSpec((1, E), lambda i: (0, 0))
    return pl.pallas_call(
        _qkv_body,
        grid=(NBLK,),
        in_specs=[row, vec, vec, full, full, full, vec, vec, vec],
        out_specs=[row, row, row],
        out_shape=[jax.ShapeDtypeStruct((S, E), jnp.float32)] * 3,
    )(x, g, b, wq, wk, wv, bq, bk, bv)


def _attn_body(q_ref, k_ref, v_ref, o_ref):
    i = pl.program_id(1)
    qb = q_ref[0].astype(jnp.bfloat16)
    kb = k_ref[0].astype(jnp.bfloat16)
    s = lax.dot_general(qb, kb, (((1,), (1,)), ((), ())),
                        preferred_element_type=jnp.float32)
    s = s * (1.0 / math.sqrt(E))
    rows = i * BT + lax.broadcasted_iota(jnp.int32, (BT, S), 0)
    cols = lax.broadcasted_iota(jnp.int32, (BT, S), 1)
    s = jnp.where(cols <= rows, s, NEG)
    m = jnp.max(s, axis=1, keepdims=True)
    p = jnp.exp(s - m)
    l = jnp.sum(p, axis=1, keepdims=True)
    pv = jnp.dot(p.astype(jnp.bfloat16), v_ref[0].astype(jnp.bfloat16),
                 preferred_element_type=jnp.float32)
    o_ref[0] = pv / l


def _attn(q, k, v):
    qspec = pl.BlockSpec((1, BT, HS), lambda h, i: (h, i, 0))
    kspec = pl.BlockSpec((1, S, HS), lambda h, i: (h, 0, 0))
    return pl.pallas_call(
        _attn_body,
        grid=(H, NBLK),
        in_specs=[qspec, kspec, kspec],
        out_specs=qspec,
        out_shape=jax.ShapeDtypeStruct((H, S, HS), jnp.float32),
    )(q, k, v)


def _outproj_body(o_ref, wo_ref, bo_ref, xr_ref, g2_ref, b2_ref, rw_ref,
                  rb_ref, x1_ref, y_ref, rs_ref):
    x1 = (jnp.dot(o_ref[...], wo_ref[...], preferred_element_type=jnp.float32)
          + bo_ref[...] + xr_ref[...])
    x1_ref[...] = x1
    y = _ln_rows(x1, g2_ref[...], b2_ref[...])
    y_ref[...] = y
    rs_ref[...] = jnp.dot(y, rw_ref[...], preferred_element_type=jnp.float32) + rb_ref[...]


def _outproj(o, wo, bo, xr, g2, b2, rw, rb):
    row = pl.BlockSpec((BT, E), lambda i: (i, 0))
    full = pl.BlockSpec((E, E), lambda i: (0, 0))
    vec = pl.BlockSpec((1, E), lambda i: (0, 0))
    return pl.pallas_call(
        _outproj_body,
        grid=(NBLK,),
        in_specs=[row, full, vec, row, vec, vec,
                  pl.BlockSpec((E, EX), lambda i: (0, 0)),
                  pl.BlockSpec((1, EX), lambda i: (0, 0))],
        out_specs=[row, row, pl.BlockSpec((BT, EX), lambda i: (i, 0))],
        out_shape=[jax.ShapeDtypeStruct((S, E), jnp.float32),
                   jax.ShapeDtypeStruct((S, E), jnp.float32),
                   jax.ShapeDtypeStruct((S, EX), jnp.float32)],
    )(o, wo, bo, xr, g2, b2, rw, rb)


def _ffn_body(be_ref, ba_ref, y_ref, w1_ref, b1_ref, w2_ref, b2_ref, out_ref):
    b = pl.program_id(0)

    @pl.when(ba_ref[b] == 1)
    def _():
        y = y_ref[...].astype(jnp.bfloat16)
        h = jnp.dot(y, w1_ref[0], preferred_element_type=jnp.float32) + b1_ref[0]
        h = jnp.maximum(h, 0.0).astype(jnp.bfloat16)
        out_ref[...] = jnp.dot(h, w2_ref[0], preferred_element_type=jnp.float32) + b2_ref[0]


def _ffn(be, ba, y_rows, ew1, eb1, ew2, eb2):
    grid_spec = pltpu.PrefetchScalarGridSpec(
        num_scalar_prefetch=2,
        grid=(GBLK,),
        in_specs=[
            pl.BlockSpec((BT, E), lambda b, be, ba: (jnp.where(ba[b] == 1, b, 0), 0)),
            pl.BlockSpec((1, E, FF), lambda b, be, ba: (be[b], 0, 0)),
            pl.BlockSpec((1, 1, FF), lambda b, be, ba: (be[b], 0, 0)),
            pl.BlockSpec((1, FF, E), lambda b, be, ba: (be[b], 0, 0)),
            pl.BlockSpec((1, 1, E), lambda b, be, ba: (be[b], 0, 0)),
        ],
        out_specs=pl.BlockSpec((BT, E), lambda b, be, ba: (b, 0)),
    )
    return pl.pallas_call(
        _ffn_body,
        grid_spec=grid_spec,
        out_shape=jax.ShapeDtypeStruct((CAP, E), jnp.float32),
    )(be, ba, y_rows, ew1, eb1, ew2, eb2)


def _combine_body(x1_ref, g1_ref, g2_ref, w1_ref, w2_ref, o_ref):
    w1 = w1_ref[:, 0:1]
    w2 = w2_ref[:, 0:1]
    o_ref[...] = x1_ref[...] + g1_ref[...] * w1 + g2_ref[...] * w2


def _combine(x1, g1, g2, w1b, w2b):
    row = pl.BlockSpec((BT, E), lambda i: (i, 0))
    wspec = pl.BlockSpec((BT, 128), lambda i: (i, 0))
    return pl.pallas_call(
        _combine_body,
        grid=(NBLK,),
        in_specs=[row, row, row, wspec, wspec],
        out_specs=row,
        out_shape=jax.ShapeDtypeStruct((S, E), jnp.float32),
    )(x1, g1, g2, w1b, w2b)


def _lnf_body(x_ref, g_ref, b_ref, o_ref):
    o_ref[...] = _ln_rows(x_ref[...], g_ref[...], b_ref[...])


def _lnf(x, g, b):
    row = pl.BlockSpec((BT, E), lambda i: (i, 0))
    vec = pl.BlockSpec((1, E), lambda i: (0, 0))
    return pl.pallas_call(
        _lnf_body,
        grid=(NBLK,),
        in_specs=[row, vec, vec],
        out_specs=row,
        out_shape=jax.ShapeDtypeStruct((S, E), jnp.float32),
    )(x, g, b)


_BR = 512
_BV = 512


def _lmhead_body(x_ref, w_ref, b_ref, o_ref):
    xb = x_ref[...].astype(jnp.bfloat16)
    o_ref[...] = (jnp.dot(xb, w_ref[...], preferred_element_type=jnp.float32)
                  + b_ref[...])


def _lmhead(x, w, b):
    return pl.pallas_call(
        _lmhead_body,
        grid=(S // _BR, V // _BV),
        in_specs=[pl.BlockSpec((_BR, E), lambda i, j: (i, 0)),
                  pl.BlockSpec((E, _BV), lambda i, j: (0, j)),
                  pl.BlockSpec((1, _BV), lambda i, j: (0, j))],
        out_specs=pl.BlockSpec((_BR, _BV), lambda i, j: (i, j)),
        out_shape=jax.ShapeDtypeStruct((S, V), jnp.float32),
    )(x, w, b)


@functools.lru_cache(maxsize=None)
def _make_sc_gather(nrows_table, ncols, nrows_out):
    info = plsc.get_sparse_core_info()
    nw = info.num_cores * info.num_subcores
    b_per_w = nrows_out // nw
    ch = min(b_per_w, 64)
    n_chunks = b_per_w // ch
    mesh = plsc.VectorSubcoreMesh(core_axis_name="c", subcore_axis_name="s")

    @functools.partial(
        pl.kernel, mesh=mesh,
        out_type=jax.ShapeDtypeStruct((nrows_out, ncols), jnp.float32),
        compiler_params=pltpu.CompilerParams(needs_layout_passes=False),
        scratch_types=[pltpu.VMEM((ch,), jnp.int32),
                       pltpu.VMEM((ch, ncols), jnp.float32),
                       pltpu.SemaphoreType.DMA],
    )
    def k(table_hbm, idx_hbm, out_hbm, idx_v, rows_v, sem):
        wid = lax.axis_index("s") * info.num_cores + lax.axis_index("c")
        base = wid * b_per_w
        for c in range(n_chunks):
            off = base + c * ch
            pltpu.sync_copy(idx_hbm.at[pl.ds(off, ch)], idx_v)
            pltpu.async_copy(table_hbm.at[idx_v], rows_v, sem).wait()
            pltpu.sync_copy(rows_v, out_hbm.at[pl.ds(off, ch)])

    return k


def _sc_gather(table, idx):
    k = _make_sc_gather(table.shape[0], table.shape[1], idx.shape[0])
    return k(table, idx)


def _sc_meta_body(rs_hbm, src_hbm, pp1_hbm, pp2_hbm, w1_hbm, w2_hbm, be_hbm,
                  ba_hbm, rsl, e1l, e2l, w1l, w2l, ecl, seg, crow, cuml, pp1l,
                  pp2l, z256, bel, bal, ec_s, cum_s, sem):
    core = lax.axis_index("c")
    s = lax.axis_index("s")

    @pl.when(core == 0)
    def _():
        iota = lax.broadcasted_iota(jnp.int32, (16,), 0)

        pltpu.sync_copy(rs_hbm, rsl)

        def pa(c, carry):
            gsl = pl.ds(s * 128 + c * 16, 16)
            r = [rsl[e, gsl] for e in range(EX)]
            m1 = r[0]
            for e in range(1, EX):
                m1 = jnp.maximum(m1, r[e])
            i1 = jnp.zeros((16,), jnp.int32)
            for e in range(EX - 1, -1, -1):
                i1 = jnp.where(r[e] == m1, e, i1)
            r2 = [jnp.where(i1 == e, NEG, r[e]) for e in range(EX)]
            m2 = r2[0]
            for e in range(1, EX):
                m2 = jnp.maximum(m2, r2[e])
            i2 = jnp.zeros((16,), jnp.int32)
            for e in range(EX - 1, -1, -1):
                i2 = jnp.where(r2[e] == m2, e, i2)
            ew = jnp.exp(m2 - m1)
            den = 1.0 + ew
            lsl = pl.ds(c * 16, 16)
            e1l[lsl] = i1
            e2l[lsl] = i2
            w1l[lsl] = 1.0 / den
            w2l[lsl] = ew / den
            return carry

        lax.fori_loop(0, 8, pa, jnp.int32(0))
        pltpu.sync_copy(w1l, w1_hbm.at[pl.ds(s * 128, 128)])
        pltpu.sync_copy(w2l, w2_hbm.at[pl.ds(s * 128, 128)])
        pltpu.sync_copy(e1l, ec_s.at[0, pl.ds(s * 128, 128)])
        pltpu.sync_copy(e2l, ec_s.at[1, pl.ds(s * 128, 128)])
        plsc.subcore_barrier()

        @pl.when(s < EX)
        def _b1():
            pltpu.sync_copy(ec_s, ecl)

            def cb(g, acc):
                rowvec = jnp.zeros((16,), jnp.int32)
                for cc in range(16):
                    jsl = pl.ds(g * 256 + cc * 16, 16)
                    rowvec = jnp.where(iota == cc, acc, rowvec)
                    m = (jnp.where(ecl[0, jsl] == s, 1, 0)
                         + jnp.where(ecl[1, jsl] == s, 1, 0))
                    acc = acc + jnp.sum(m)
                crow[pl.ds(g * 16, 16)] = rowvec
                return acc

            tot = lax.fori_loop(0, 8, cb, jnp.int32(0))
            crow[pl.ds(128, 16)] = jnp.zeros((16,), jnp.int32) + tot
            def zpad(i, c):
                crow[pl.ds(144 + i * 16, 16)] = jnp.zeros((16,), jnp.int32)
                return c

            lax.fori_loop(0, 7, zpad, jnp.int32(0))
            pltpu.sync_copy(crow, cum_s.at[0, pl.ds(s * 256, 256)])

        plsc.subcore_barrier()

        pltpu.sync_copy(cum_s, cuml)
        bases = []
        tots = []
        acc = jnp.int32(0)
        for e in range(EX):
            bases.append(acc)
            tote = cuml[0, pl.ds(e * 256 + 128, 16)][0]
            tots.append(tote)
            acc = acc + ((tote + (BT - 1)) // BT) * BT
        total = acc

        @pl.when(s < EX)
        def _b2():
            def zb(i, c):
                seg[pl.ds(i * 16, 16)] = jnp.zeros((16,), jnp.int32)
                return c

            lax.fori_loop(0, (S + 16) // 16, zb, jnp.int32(0))

            def sb(j, cur):
                tv = iota + j * 16
                jsl = pl.ds(j * 16, 16)
                m1 = ecl[0, jsl] == s
                plsc.store_compressed(seg.at[pl.ds(cur, 16)], tv, mask=m1)
                cur = cur + jnp.sum(jnp.where(m1, 1, 0))
                m2 = ecl[1, jsl] == s
                plsc.store_compressed(seg.at[pl.ds(cur, 16)], tv, mask=m2)
                cur = cur + jnp.sum(jnp.where(m2, 1, 0))
                return cur

            tot_self = lax.fori_loop(0, 128, sb, jnp.int32(0))
            mybase = jnp.int32(0)
            for e in range(EX):
                mybase = jnp.where(s == e, bases[e], mybase)
            for i in range(EX):
                @pl.when(i * BT < tot_self)
                def _cp(i=i):
                    off = pl.multiple_of(mybase + i * BT, BT)
                    pltpu.sync_copy(seg.at[pl.ds(i * BT, BT)],
                                    src_hbm.at[pl.ds(off, BT)])

        @pl.when(s == EX)
        def _tz():
            for i in range(16):
                z256[pl.ds(i * 16, 16)] = jnp.zeros((16,), jnp.int32)
            for i in range(GBLK):
                @pl.when(i * BT >= total)
                def _z(i=i):
                    pltpu.sync_copy(z256, src_hbm.at[pl.ds(i * BT, BT)])

        @pl.when(s == EX + 1)
        def _tb():
            for jb in range(2):
                bv = iota + jb * 16
                rowstart = bv * BT
                bex = jnp.zeros((16,), jnp.int32)
                for e in range(1, EX):
                    bex = jnp.where(rowstart >= bases[e], e, bex)
                bact = jnp.where(rowstart < total, 1, 0)
                bel[pl.ds(jb * 16, 16)] = bex
                bal[pl.ds(jb * 16, 16)] = bact
            pltpu.sync_copy(bel, be_hbm)
            pltpu.sync_copy(bal, ba_hbm)

        def pc(c, carry):
            lane = (s % 2) * 8 + c
            blk = (s // 2) * 16
            lsl = pl.ds(c * 16, 16)
            e1 = e1l[lsl]
            e2 = e2l[lsl]
            pos1 = jnp.zeros((16,), jnp.int32)
            pos2 = jnp.zeros((16,), jnp.int32)
            for e in range(EX):
                m1 = e1 == e
                mi1 = jnp.where(m1, 1, 0)
                cs1 = plsc.cumsum(mi1)
                n1 = jnp.sum(mi1)
                m2 = e2 == e
                mi2 = jnp.where(m2, 1, 0)
                cs2 = plsc.cumsum(mi2)
                cumv = cuml[0, pl.ds(e * 256 + blk, 16)]
                cumej = jnp.sum(jnp.where(iota == lane, cumv, 0))
                start = bases[e] + cumej
                pos1 = jnp.where(m1, start + cs1 - 1, pos1)
                pos2 = jnp.where(m2, start + n1 + cs2 - 1, pos2)
            pp1l[lsl] = pos1
            pp2l[lsl] = pos2
            return carry

        lax.fori_loop(0, 8, pc, jnp.int32(0))
        pltpu.sync_copy(pp1l, pp1_hbm.at[pl.ds(s * 128, 128)])
        pltpu.sync_copy(pp2l, pp2_hbm.at[pl.ds(s * 128, 128)])


@functools.lru_cache(maxsize=None)
def _make_sc_meta():
    mesh = plsc.VectorSubcoreMesh(core_axis_name="c", subcore_axis_name="s")
    return functools.partial(
        pl.kernel, mesh=mesh,
        compiler_params=pltpu.CompilerParams(needs_layout_passes=False),
        out_type=[jax.ShapeDtypeStruct((CAP,), jnp.int32),
                  jax.ShapeDtypeStruct((S,), jnp.int32),
                  jax.ShapeDtypeStruct((S,), jnp.int32),
                  jax.ShapeDtypeStruct((S,), jnp.float32),
                  jax.ShapeDtypeStruct((S,), jnp.float32),
                  jax.ShapeDtypeStruct((32,), jnp.int32),
                  jax.ShapeDtypeStruct((32,), jnp.int32)],
        scratch_types=[pltpu.VMEM((EX, S), jnp.float32),
                       pltpu.VMEM((128,), jnp.int32),
                       pltpu.VMEM((128,), jnp.int32),
                       pltpu.VMEM((128,), jnp.float32),
                       pltpu.VMEM((128,), jnp.float32),
                       pltpu.VMEM((2, S), jnp.int32),
                       pltpu.VMEM((S + 16,), jnp.int32),
                       pltpu.VMEM((256,), jnp.int32),
                       pltpu.VMEM((1, EX * 256), jnp.int32),
                       pltpu.VMEM((128,), jnp.int32),
                       pltpu.VMEM((128,), jnp.int32),
                       pltpu.VMEM((BT,), jnp.int32),
                       pltpu.VMEM((32,), jnp.int32),
                       pltpu.VMEM((32,), jnp.int32),
                       pltpu.VMEM_SHARED((2, S), jnp.int32),
                       pltpu.VMEM_SHARED((1, EX * 256), jnp.int32),
                       pltpu.SemaphoreType.DMA],
    )(_sc_meta_body)


def _sc_meta(rs_t):
    return _make_sc_meta()(rs_t)


def kernel(inputs, tok_emb, pos_emb, ln1_g, ln1_b, wq, bq, wk, bk, wv, bv, wo,
           bo, ln2_g, ln2_b, router_w, router_b, e_w1, e_b1, e_w2, e_b2,
           lnf_g, lnf_b, out_w, out_b):
    ids = inputs.reshape(S).astype(jnp.int32)
    emb = _sc_gather(tok_emb, ids)
    x = _add2(emb, pos_emb)
    for l in range(LAYERS):
        wq_l = wq[l].transpose(1, 0, 2).reshape(E, H * HS)
        wk_l = wk[l].transpose(1, 0, 2).reshape(E, H * HS)
        wv_l = wv[l].transpose(1, 0, 2).reshape(E, H * HS)
        q, k, v = _qkv(x, ln1_g[l].reshape(1, E), ln1_b[l].reshape(1, E),
                       wq_l, wk_l, wv_l, bq[l].reshape(1, H * HS),
                       bk[l].reshape(1, H * HS), bv[l].reshape(1, H * HS))
        qh = q.reshape(S, H, HS).transpose(1, 0, 2)
        kh = k.reshape(S, H, HS).transpose(1, 0, 2)
        vh = v.reshape(S, H, HS).transpose(1, 0, 2)
        o = _attn(qh, kh, vh).transpose(1, 0, 2).reshape(S, H * HS)
        x1, y, rs = _outproj(o, wo[l], bo[l].reshape(1, E), x,
                             ln2_g[l].reshape(1, E), ln2_b[l].reshape(1, E),
                             router_w[l], router_b[l].reshape(1, EX))
        src_tok, pp1, pp2, w1, w2, be, ba = _sc_meta(rs.T)
        y_rows = _sc_gather(y, src_tok)
        out_g = _ffn(be, ba, y_rows, e_w1[l].astype(jnp.bfloat16),
                     e_b1[l].reshape(EX, 1, FF),
                     e_w2[l].astype(jnp.bfloat16),
                     e_b2[l].reshape(EX, 1, E))
        g1 = _sc_gather(out_g, pp1)
        g2 = _sc_gather(out_g, pp2)
        w1b = jnp.broadcast_to(w1[:, None], (S, 128))
        w2b = jnp.broadcast_to(w2[:, None], (S, 128))
        x = _combine(x1, g1, g2, w1b, w2b)
    xf = _lnf(x, lnf_g.reshape(1, E), lnf_b.reshape(1, E))
    logits = _lmhead(xf, out_w.astype(jnp.bfloat16), out_b.reshape(1, V))
    return logits.reshape(1, S, V)


# speedup vs baseline: 1.1722x; 1.0204x over previous
import functools
import math

import jax
import jax.numpy as jnp
from jax import lax
from jax.experimental import pallas as pl
from jax.experimental.pallas import tpu as pltpu
from jax.experimental.pallas import tpu_sc as plsc

S = 2048
E = 768
V = 8192
H = 12
HS = 64
EX = 8
FF = 3072
LAYERS = 2

BT = 256
NBLK = S // BT
CAP = S * 2 + EX * BT
GBLK = CAP // BT
NEG = -3.0e38


def _ln_rows(x, g, b):
    m = jnp.mean(x, axis=1, keepdims=True)
    xc = x - m
    var = jnp.mean(xc * xc, axis=1, keepdims=True)
    return xc * lax.rsqrt(var + 1e-5) * g + b


def _add2_body(a_ref, b_ref, o_ref):
    o_ref[...] = a_ref[...] + b_ref[...]


def _add2(a, b):
    return pl.pallas_call(
        _add2_body,
        grid=(NBLK,),
        in_specs=[pl.BlockSpec((BT, E), lambda i: (i, 0)),
                  pl.BlockSpec((BT, E), lambda i: (i, 0))],
        out_specs=pl.BlockSpec((BT, E), lambda i: (i, 0)),
        out_shape=jax.ShapeDtypeStruct((S, E), jnp.float32),
    )(a, b)


def _qkv_body(x_ref, g_ref, b_ref, wq_ref, wk_ref, wv_ref, bq_ref, bk_ref,
              bv_ref, q_ref, k_ref, v_ref):
    h = _ln_rows(x_ref[...], g_ref[...], b_ref[...])
    q_ref[...] = jnp.dot(h, wq_ref[...], preferred_element_type=jnp.float32) + bq_ref[...]
    k_ref[...] = jnp.dot(h, wk_ref[...], preferred_element_type=jnp.float32) + bk_ref[...]
    v_ref[...] = jnp.dot(h, wv_ref[...], preferred_element_type=jnp.float32) + bv_ref[...]


def _qkv(x, g, b, wq, wk, wv, bq, bk, bv):
    row = pl.BlockSpec((BT, E), lambda i: (i, 0))
    full = pl.BlockSpec((E, E), lambda i: (0, 0))
    vec = pl.BlockSpec((1, E), lambda i: (0, 0))
    return pl.pallas_call(
        _qkv_body,
        grid=(NBLK,),
        in_specs=[row, vec, vec, full, full, full, vec, vec, vec],
        out_specs=[row, row, row],
        out_shape=[jax.ShapeDtypeStruct((S, E), jnp.float32)] * 3,
    )(x, g, b, wq, wk, wv, bq, bk, bv)


def _attn_body(q_ref, k_ref, v_ref, o_ref):
    i = pl.program_id(1)
    qb = q_ref[0].astype(jnp.bfloat16)
    kb = k_ref[0].astype(jnp.bfloat16)
    s = lax.dot_general(qb, kb, (((1,), (1,)), ((), ())),
                        preferred_element_type=jnp.float32)
    s = s * (1.0 / math.sqrt(E))
    rows = i * BT + lax.broadcasted_iota(jnp.int32, (BT, S), 0)
    cols = lax.broadcasted_iota(jnp.int32, (BT, S), 1)
    s = jnp.where(cols <= rows, s, NEG)
    m = jnp.max(s, axis=1, keepdims=True)
    p = jnp.exp(s - m)
    l = jnp.sum(p, axis=1, keepdims=True)
    pv = jnp.dot(p.astype(jnp.bfloat16), v_ref[0].astype(jnp.bfloat16),
                 preferred_element_type=jnp.float32)
    o_ref[0] = pv / l


def _attn(q, k, v):
    qspec = pl.BlockSpec((1, BT, HS), lambda h, i: (h, i, 0))
    kspec = pl.BlockSpec((1, S, HS), lambda h, i: (h, 0, 0))
    return pl.pallas_call(
        _attn_body,
        grid=(H, NBLK),
        in_specs=[qspec, kspec, kspec],
        out_specs=qspec,
        out_shape=jax.ShapeDtypeStruct((H, S, HS), jnp.float32),
    )(q, k, v)


def _outproj_body(o_ref, wo_ref, bo_ref, xr_ref, g2_ref, b2_ref, rw_ref,
                  rb_ref, x1_ref, y_ref, rs_ref):
    x1 = (jnp.dot(o_ref[...], wo_ref[...], preferred_element_type=jnp.float32)
          + bo_ref[...] + xr_ref[...])
    x1_ref[...] = x1
    y = _ln_rows(x1, g2_ref[...], b2_ref[...])
    y_ref[...] = y
    rs_ref[...] = jnp.dot(y, rw_ref[...], preferred_element_type=jnp.float32) + rb_ref[...]


def _outproj(o, wo, bo, xr, g2, b2, rw, rb):
    row = pl.BlockSpec((BT, E), lambda i: (i, 0))
    full = pl.BlockSpec((E, E), lambda i: (0, 0))
    vec = pl.BlockSpec((1, E), lambda i: (0, 0))
    return pl.pallas_call(
        _outproj_body,
        grid=(NBLK,),
        in_specs=[row, full, vec, row, vec, vec,
                  pl.BlockSpec((E, EX), lambda i: (0, 0)),
                  pl.BlockSpec((1, EX), lambda i: (0, 0))],
        out_specs=[row, row, pl.BlockSpec((BT, EX), lambda i: (i, 0))],
        out_shape=[jax.ShapeDtypeStruct((S, E), jnp.float32),
                   jax.ShapeDtypeStruct((S, E), jnp.float32),
                   jax.ShapeDtypeStruct((S, EX), jnp.float32)],
    )(o, wo, bo, xr, g2, b2, rw, rb)


def _ffn_body(be_ref, ba_ref, y_ref, w1_ref, b1_ref, w2_ref, b2_ref, out_ref):
    b = pl.program_id(0)

    @pl.when(ba_ref[b] == 1)
    def _():
        y = y_ref[...].astype(jnp.bfloat16)
        h = jnp.dot(y, w1_ref[0], preferred_element_type=jnp.float32) + b1_ref[0]
        h = jnp.maximum(h, 0.0).astype(jnp.bfloat16)
        out_ref[...] = jnp.dot(h, w2_ref[0], preferred_element_type=jnp.float32) + b2_ref[0]


def _ffn(be, ba, y_rows, ew1, eb1, ew2, eb2):
    grid_spec = pltpu.PrefetchScalarGridSpec(
        num_scalar_prefetch=2,
        grid=(GBLK,),
        in_specs=[
            pl.BlockSpec((BT, E), lambda b, be, ba: (jnp.where(ba[b] == 1, b, 0), 0)),
            pl.BlockSpec((1, E, FF), lambda b, be, ba: (be[b], 0, 0)),
            pl.BlockSpec((1, 1, FF), lambda b, be, ba: (be[b], 0, 0)),
            pl.BlockSpec((1, FF, E), lambda b, be, ba: (be[b], 0, 0)),
            pl.BlockSpec((1, 1, E), lambda b, be, ba: (be[b], 0, 0)),
        ],
        out_specs=pl.BlockSpec((BT, E), lambda b, be, ba: (b, 0)),
    )
    return pl.pallas_call(
        _ffn_body,
        grid_spec=grid_spec,
        out_shape=jax.ShapeDtypeStruct((CAP, E), jnp.float32),
    )(be, ba, y_rows, ew1, eb1, ew2, eb2)


def _combine_body(x1_ref, g1_ref, g2_ref, w1_ref, w2_ref, o_ref):
    w1 = w1_ref[:, 0:1]
    w2 = w2_ref[:, 0:1]
    o_ref[...] = x1_ref[...] + g1_ref[...] * w1 + g2_ref[...] * w2


def _combine(x1, g1, g2, w1b, w2b):
    row = pl.BlockSpec((BT, E), lambda i: (i, 0))
    wspec = pl.BlockSpec((BT, 128), lambda i: (i, 0))
    return pl.pallas_call(
        _combine_body,
        grid=(NBLK,),
        in_specs=[row, row, row, wspec, wspec],
        out_specs=row,
        out_shape=jax.ShapeDtypeStruct((S, E), jnp.float32),
    )(x1, g1, g2, w1b, w2b)


def _lnf_body(x_ref, g_ref, b_ref, o_ref):
    o_ref[...] = _ln_rows(x_ref[...], g_ref[...], b_ref[...])


def _lnf(x, g, b):
    row = pl.BlockSpec((BT, E), lambda i: (i, 0))
    vec = pl.BlockSpec((1, E), lambda i: (0, 0))
    return pl.pallas_call(
        _lnf_body,
        grid=(NBLK,),
        in_specs=[row, vec, vec],
        out_specs=row,
        out_shape=jax.ShapeDtypeStruct((S, E), jnp.float32),
    )(x, g, b)


_BR = 512
_BV = 512


def _lmhead_body(x_ref, w_ref, b_ref, o_ref):
    xb = x_ref[...].astype(jnp.bfloat16)
    o_ref[...] = (jnp.dot(xb, w_ref[...], preferred_element_type=jnp.float32)
                  + b_ref[...])


def _lmhead(x, w, b):
    return pl.pallas_call(
        _lmhead_body,
        grid=(S // _BR, V // _BV),
        in_specs=[pl.BlockSpec((_BR, E), lambda i, j: (i, 0)),
                  pl.BlockSpec((E, _BV), lambda i, j: (0, j)),
                  pl.BlockSpec((1, _BV), lambda i, j: (0, j))],
        out_specs=pl.BlockSpec((_BR, _BV), lambda i, j: (i, j)),
        out_shape=jax.ShapeDtypeStruct((S, V), jnp.float32),
    )(x, w, b)


@functools.lru_cache(maxsize=None)
def _make_sc_gather(nrows_table, ncols, nrows_out):
    info = plsc.get_sparse_core_info()
    nw = info.num_cores * info.num_subcores
    b_per_w = nrows_out // nw
    ch = min(b_per_w, 64)
    n_chunks = b_per_w // ch
    mesh = plsc.VectorSubcoreMesh(core_axis_name="c", subcore_axis_name="s")

    @functools.partial(
        pl.kernel, mesh=mesh,
        out_type=jax.ShapeDtypeStruct((nrows_out, ncols), jnp.float32),
        compiler_params=pltpu.CompilerParams(needs_layout_passes=False),
        scratch_types=[pltpu.VMEM((ch,), jnp.int32),
                       pltpu.VMEM((ch, ncols), jnp.float32),
                       pltpu.SemaphoreType.DMA],
    )
    def k(table_hbm, idx_hbm, out_hbm, idx_v, rows_v, sem):
        wid = lax.axis_index("s") * info.num_cores + lax.axis_index("c")
        base = wid * b_per_w
        for c in range(n_chunks):
            off = base + c * ch
            pltpu.sync_copy(idx_hbm.at[pl.ds(off, ch)], idx_v)
            pltpu.async_copy(table_hbm.at[idx_v], rows_v, sem).wait()
            pltpu.sync_copy(rows_v, out_hbm.at[pl.ds(off, ch)])

    return k


def _sc_gather(table, idx):
    k = _make_sc_gather(table.shape[0], table.shape[1], idx.shape[0])
    return k(table, idx)


@functools.lru_cache(maxsize=None)
def _make_sc_embed():
    info = plsc.get_sparse_core_info()
    nw = info.num_cores * info.num_subcores
    b_per_w = S // nw
    mesh = plsc.VectorSubcoreMesh(core_axis_name="c", subcore_axis_name="s")

    @functools.partial(
        pl.kernel, mesh=mesh,
        out_type=jax.ShapeDtypeStruct((S, E), jnp.float32),
        compiler_params=pltpu.CompilerParams(needs_layout_passes=False),
        scratch_types=[pltpu.VMEM((b_per_w,), jnp.int32),
                       pltpu.VMEM((16, E), jnp.float32),
                       pltpu.VMEM((16, E), jnp.float32),
                       pltpu.SemaphoreType.DMA],
    )
    def k(tab_hbm, ids_hbm, pos_hbm, out_hbm, idc, gc, pc, sem):
        wid = lax.axis_index("s") * info.num_cores + lax.axis_index("c")
        base = wid * b_per_w
        pltpu.sync_copy(ids_hbm.at[pl.ds(base, b_per_w)], idc)
        for c in range(b_per_w // 16):
            rb = base + c * 16
            pltpu.async_copy(tab_hbm.at[idc.at[pl.ds(c * 16, 16)]], gc,
                             sem).wait()
            pltpu.sync_copy(pos_hbm.at[pl.ds(rb, 16)], pc)

            def ab(kk, carry):
                ksl = pl.ds(kk * 16, 16)
                for r in range(16):
                    gc[r, ksl] = gc[r, ksl] + pc[r, ksl]
                return carry

            lax.fori_loop(0, E // 16, ab, jnp.int32(0))
            pltpu.sync_copy(gc, out_hbm.at[pl.ds(rb, 16)])

    return k


def _sc_embed(tab, ids, pos):
    return _make_sc_embed()(tab, ids, pos)


@functools.lru_cache(maxsize=None)
def _make_sc_combine():
    info = plsc.get_sparse_core_info()
    nw = info.num_cores * info.num_subcores
    b_per_w = S // nw
    mesh = plsc.VectorSubcoreMesh(core_axis_name="c", subcore_axis_name="s")

    @functools.partial(
        pl.kernel, mesh=mesh,
        out_type=jax.ShapeDtypeStruct((S, E), jnp.float32),
        compiler_params=pltpu.CompilerParams(needs_layout_passes=False),
        scratch_types=[pltpu.VMEM((b_per_w,), jnp.int32),
                       pltpu.VMEM((b_per_w,), jnp.int32),
                       pltpu.VMEM((b_per_w,), jnp.float32),
                       pltpu.VMEM((b_per_w,), jnp.float32),
                       pltpu.VMEM((16, E), jnp.float32),
                       pltpu.VMEM((16, E), jnp.float32),
                       pltpu.VMEM((16, E), jnp.float32),
                       pltpu.SemaphoreType.DMA,
                       pltpu.SemaphoreType.DMA],
    )
    def k(x1_hbm, og_hbm, pp1_hbm, pp2_hbm, w1_hbm, w2_hbm, out_hbm,
          ppc1, ppc2, wc1, wc2, xc, g1c, g2c, sem1, sem2):
        wid = lax.axis_index("s") * info.num_cores + lax.axis_index("c")
        base = wid * b_per_w
        pltpu.sync_copy(pp1_hbm.at[pl.ds(base, b_per_w)], ppc1)
        pltpu.sync_copy(pp2_hbm.at[pl.ds(base, b_per_w)], ppc2)
        pltpu.sync_copy(w1_hbm.at[pl.ds(base, b_per_w)], wc1)
        pltpu.sync_copy(w2_hbm.at[pl.ds(base, b_per_w)], wc2)
        for c in range(b_per_w // 16):
            rb = base + c * 16
            csl = pl.ds(c * 16, 16)
            h1 = pltpu.async_copy(og_hbm.at[ppc1.at[csl]], g1c, sem1)
            h2 = pltpu.async_copy(og_hbm.at[ppc2.at[csl]], g2c, sem2)
            pltpu.sync_copy(x1_hbm.at[pl.ds(rb, 16)], xc)
            h1.wait()
            h2.wait()
            w1v = wc1[csl]
            w2v = wc2[csl]
            w1s = [w1v[r] for r in range(16)]
            w2s = [w2v[r] for r in range(16)]

            def ab(kk, carry):
                ksl = pl.ds(kk * 16, 16)
                for r in range(16):
                    xc[r, ksl] = (xc[r, ksl] + w1s[r] * g1c[r, ksl]
                                  + w2s[r] * g2c[r, ksl])
                return carry

            lax.fori_loop(0, E // 16, ab, jnp.int32(0))
            pltpu.sync_copy(xc, out_hbm.at[pl.ds(rb, 16)])

    return k


def _sc_combine(x1, og, pp1, pp2, w1, w2):
    return _make_sc_combine()(x1, og, pp1, pp2, w1, w2)


def _sc_meta_body(rs_hbm, src_hbm, pp1_hbm, pp2_hbm, w1_hbm, w2_hbm, be_hbm,
                  ba_hbm, rsl, e1l, e2l, w1l, w2l, ecl, seg, crow, cuml, pp1l,
                  pp2l, z256, bel, bal, ec_s, cum_s, sem):
    core = lax.axis_index("c")
    s = lax.axis_index("s")

    @pl.when(core == 0)
    def _():
        iota = lax.broadcasted_iota(jnp.int32, (16,), 0)

        pltpu.sync_copy(rs_hbm, rsl)

        def pa(c, carry):
            gsl = pl.ds(s * 128 + c * 16, 16)
            r = [rsl[e, gsl] for e in range(EX)]
            m1 = r[0]
            for e in range(1, EX):
                m1 = jnp.maximum(m1, r[e])
            i1 = jnp.zeros((16,), jnp.int32)
            for e in range(EX - 1, -1, -1):
                i1 = jnp.where(r[e] == m1, e, i1)
            r2 = [jnp.where(i1 == e, NEG, r[e]) for e in range(EX)]
            m2 = r2[0]
            for e in range(1, EX):
                m2 = jnp.maximum(m2, r2[e])
            i2 = jnp.zeros((16,), jnp.int32)
            for e in range(EX - 1, -1, -1):
                i2 = jnp.where(r2[e] == m2, e, i2)
            ew = jnp.exp(m2 - m1)
            den = 1.0 + ew
            lsl = pl.ds(c * 16, 16)
            e1l[lsl] = i1
            e2l[lsl] = i2
            w1l[lsl] = 1.0 / den
            w2l[lsl] = ew / den
            return carry

        lax.fori_loop(0, 8, pa, jnp.int32(0))
        pltpu.sync_copy(w1l, w1_hbm.at[pl.ds(s * 128, 128)])
        pltpu.sync_copy(w2l, w2_hbm.at[pl.ds(s * 128, 128)])
        pltpu.sync_copy(e1l, ec_s.at[0, pl.ds(s * 128, 128)])
        pltpu.sync_copy(e2l, ec_s.at[1, pl.ds(s * 128, 128)])
        plsc.subcore_barrier()

        @pl.when(s < EX)
        def _b1():
            pltpu.sync_copy(ec_s, ecl)

            def cb(g, acc):
                rowvec = jnp.zeros((16,), jnp.int32)
                for cc in range(16):
                    jsl = pl.ds(g * 256 + cc * 16, 16)
                    rowvec = jnp.where(iota == cc, acc, rowvec)
                    m = (jnp.where(ecl[0, jsl] == s, 1, 0)
                         + jnp.where(ecl[1, jsl] == s, 1, 0))
                    acc = acc + jnp.sum(m)
                crow[pl.ds(g * 16, 16)] = rowvec
                return acc

            tot = lax.fori_loop(0, 8, cb, jnp.int32(0))
            crow[pl.ds(128, 16)] = jnp.zeros((16,), jnp.int32) + tot
            def zpad(i, c):
                crow[pl.ds(144 + i * 16, 16)] = jnp.zeros((16,), jnp.int32)
                return c

            lax.fori_loop(0, 7, zpad, jnp.int32(0))
            pltpu.sync_copy(crow, cum_s.at[0, pl.ds(s * 256, 256)])

        plsc.subcore_barrier()

        pltpu.sync_copy(cum_s, cuml)
        bases = []
        tots = []
        acc = jnp.int32(0)
        for e in range(EX):
            bases.append(acc)
            tote = cuml[0, pl.ds(e * 256 + 128, 16)][0]
            tots.append(tote)
            acc = acc + ((tote + (BT - 1)) // BT) * BT
        total = acc

        @pl.when(s < EX)
        def _b2():
            def zb(i, c):
                seg[pl.ds(i * 16, 16)] = jnp.zeros((16,), jnp.int32)
                return c

            lax.fori_loop(0, (S + 16) // 16, zb, jnp.int32(0))

            def sb(j, cur):
                tv = iota + j * 16
                jsl = pl.ds(j * 16, 16)
                m1 = ecl[0, jsl] == s
                plsc.store_compressed(seg.at[pl.ds(cur, 16)], tv, mask=m1)
                cur = cur + jnp.sum(jnp.where(m1, 1, 0))
                m2 = ecl[1, jsl] == s
                plsc.store_compressed(seg.at[pl.ds(cur, 16)], tv, mask=m2)
                cur = cur + jnp.sum(jnp.where(m2, 1, 0))
                return cur

            tot_self = lax.fori_loop(0, 128, sb, jnp.int32(0))
            mybase = jnp.int32(0)
            for e in range(EX):
                mybase = jnp.where(s == e, bases[e], mybase)
            for i in range(EX):
                @pl.when(i * BT < tot_self)
                def _cp(i=i):
                    off = pl.multiple_of(mybase + i * BT, BT)
                    pltpu.sync_copy(seg.at[pl.ds(i * BT, BT)],
                                    src_hbm.at[pl.ds(off, BT)])

        @pl.when(s == EX)
        def _tz():
            for i in range(16):
                z256[pl.ds(i * 16, 16)] = jnp.zeros((16,), jnp.int32)
            for i in range(GBLK):
                @pl.when(i * BT >= total)
                def _z(i=i):
                    pltpu.sync_copy(z256, src_hbm.at[pl.ds(i * BT, BT)])

        @pl.when(s == EX + 1)
        def _tb():
            for jb in range(2):
                bv = iota + jb * 16
                rowstart = bv * BT
                bex = jnp.zeros((16,), jnp.int32)
                for e in range(1, EX):
                    bex = jnp.where(rowstart >= bases[e], e, bex)
                bact = jnp.where(rowstart < total, 1, 0)
                bel[pl.ds(jb * 16, 16)] = bex
                bal[pl.ds(jb * 16, 16)] = bact
            pltpu.sync_copy(bel, be_hbm)
            pltpu.sync_copy(bal, ba_hbm)

        def pc(c, carry):
            lane = (s % 2) * 8 + c
            blk = (s // 2) * 16
            lsl = pl.ds(c * 16, 16)
            e1 = e1l[lsl]
            e2 = e2l[lsl]
            pos1 = jnp.zeros((16,), jnp.int32)
            pos2 = jnp.zeros((16,), jnp.int32)
            for e in range(EX):
                m1 = e1 == e
                mi1 = jnp.where(m1, 1, 0)
                cs1 = plsc.cumsum(mi1)
                n1 = jnp.sum(mi1)
                m2 = e2 == e
                mi2 = jnp.where(m2, 1, 0)
                cs2 = plsc.cumsum(mi2)
                cumv = cuml[0, pl.ds(e * 256 + blk, 16)]
                cumej = jnp.sum(jnp.where(iota == lane, cumv, 0))
                start = bases[e] + cumej
                pos1 = jnp.where(m1, start + cs1 - 1, pos1)
                pos2 = jnp.where(m2, start + n1 + cs2 - 1, pos2)
            pp1l[lsl] = pos1
            pp2l[lsl] = pos2
            return carry

        lax.fori_loop(0, 8, pc, jnp.int32(0))
        pltpu.sync_copy(pp1l, pp1_hbm.at[pl.ds(s * 128, 128)])
        pltpu.sync_copy(pp2l, pp2_hbm.at[pl.ds(s * 128, 128)])


@functools.lru_cache(maxsize=None)
def _make_sc_meta():
    mesh = plsc.VectorSubcoreMesh(core_axis_name="c", subcore_axis_name="s")
    return functools.partial(
        pl.kernel, mesh=mesh,
        compiler_params=pltpu.CompilerParams(needs_layout_passes=False),
        out_type=[jax.ShapeDtypeStruct((CAP,), jnp.int32),
                  jax.ShapeDtypeStruct((S,), jnp.int32),
                  jax.ShapeDtypeStruct((S,), jnp.int32),
                  jax.ShapeDtypeStruct((S,), jnp.float32),
                  jax.ShapeDtypeStruct((S,), jnp.float32),
                  jax.ShapeDtypeStruct((32,), jnp.int32),
                  jax.ShapeDtypeStruct((32,), jnp.int32)],
        scratch_types=[pltpu.VMEM((EX, S), jnp.float32),
                       pltpu.VMEM((128,), jnp.int32),
                       pltpu.VMEM((128,), jnp.int32),
                       pltpu.VMEM((128,), jnp.float32),
                       pltpu.VMEM((128,), jnp.float32),
                       pltpu.VMEM((2, S), jnp.int32),
                       pltpu.VMEM((S + 16,), jnp.int32),
                       pltpu.VMEM((256,), jnp.int32),
                       pltpu.VMEM((1, EX * 256), jnp.int32),
                       pltpu.VMEM((128,), jnp.int32),
                       pltpu.VMEM((128,), jnp.int32),
                       pltpu.VMEM((BT,), jnp.int32),
                       pltpu.VMEM((32,), jnp.int32),
                       pltpu.VMEM((32,), jnp.int32),
                       pltpu.VMEM_SHARED((2, S), jnp.int32),
                       pltpu.VMEM_SHARED((1, EX * 256), jnp.int32),
                       pltpu.SemaphoreType.DMA],
    )(_sc_meta_body)


def _sc_meta(rs_t):
    return _make_sc_meta()(rs_t)


def kernel(inputs, tok_emb, pos_emb, ln1_g, ln1_b, wq, bq, wk, bk, wv, bv, wo,
           bo, ln2_g, ln2_b, router_w, router_b, e_w1, e_b1, e_w2, e_b2,
           lnf_g, lnf_b, out_w, out_b):
    ids = inputs.reshape(S).astype(jnp.int32)
    x = _sc_embed(tok_emb, ids, pos_emb)
    for l in range(LAYERS):
        wq_l = wq[l].transpose(1, 0, 2).reshape(E, H * HS)
        wk_l = wk[l].transpose(1, 0, 2).reshape(E, H * HS)
        wv_l = wv[l].transpose(1, 0, 2).reshape(E, H * HS)
        q, k, v = _qkv(x, ln1_g[l].reshape(1, E), ln1_b[l].reshape(1, E),
                       wq_l, wk_l, wv_l, bq[l].reshape(1, H * HS),
                       bk[l].reshape(1, H * HS), bv[l].reshape(1, H * HS))
        qh = q.reshape(S, H, HS).transpose(1, 0, 2)
        kh = k.reshape(S, H, HS).transpose(1, 0, 2)
        vh = v.reshape(S, H, HS).transpose(1, 0, 2)
        o = _attn(qh, kh, vh).transpose(1, 0, 2).reshape(S, H * HS)
        x1, y, rs = _outproj(o, wo[l], bo[l].reshape(1, E), x,
                             ln2_g[l].reshape(1, E), ln2_b[l].reshape(1, E),
                             router_w[l], router_b[l].reshape(1, EX))
        src_tok, pp1, pp2, w1, w2, be, ba = _sc_meta(rs.T)
        y_rows = _sc_gather(y, src_tok)
        out_g = _ffn(be, ba, y_rows, e_w1[l].astype(jnp.bfloat16),
                     e_b1[l].reshape(EX, 1, FF),
                     e_w2[l].astype(jnp.bfloat16),
                     e_b2[l].reshape(EX, 1, E))
        x = _sc_combine(x1, out_g, pp1, pp2, w1, w2)
    xf = _lnf(x, lnf_g.reshape(1, E), lnf_b.reshape(1, E))
    logits = _lmhead(xf, out_w.astype(jnp.bfloat16), out_b.reshape(1, V))
    return logits.reshape(1, S, V)
